# Initial kernel scaffold; baseline (speedup 1.0000x reference)
import jax
import jax.numpy as jnp
from jax.experimental import pallas as pl


def kernel(x, gate_w, gate_b, expert_w, expert_b):
    raise NotImplementedError("write your pallas kernel here")


# speedup vs baseline: 5.7517x; 5.7517x over previous
import functools

import jax
import jax.numpy as jnp
from jax.experimental import pallas as pl


def _fused_body(nexp, x_ref, gw_ref, gb_ref, ew_ref, eb_ref, topw_ref, out_ref):
    xx = x_ref[...]
    bt = xx.shape[0]
    logits = jnp.dot(xx, gw_ref[...], preferred_element_type=jnp.float32)
    logits = logits + gb_ref[...]
    lanes = jax.lax.broadcasted_iota(jnp.int32, logits.shape, 1)
    logits = jnp.where(lanes < nexp, logits, -jnp.inf)
    m = jnp.max(logits, axis=1, keepdims=True)
    ex = jnp.exp(logits - m)
    s = jnp.sum(ex, axis=1, keepdims=True)
    w = ex / s
    m1 = jnp.max(w, axis=1, keepdims=True)
    i1 = jnp.min(jnp.where(w == m1, lanes, 128), axis=1, keepdims=True)
    w2 = jnp.where(lanes == i1, -1.0, w)
    m2 = jnp.max(w2, axis=1, keepdims=True)
    i2 = jnp.min(jnp.where(w2 == m2, lanes, 128), axis=1, keepdims=True)
    topw_ref[...] = jnp.concatenate([m1, m2], axis=1)
    xb = xx.astype(jnp.bfloat16)
    acc1 = jnp.zeros((bt, out_ref.shape[2]), jnp.float32)
    acc2 = jnp.zeros((bt, out_ref.shape[2]), jnp.float32)
    for e in range(nexp):
        oe = jnp.dot(xb, ew_ref[e], preferred_element_type=jnp.float32)
        oe = oe + eb_ref[e][None, :]
        acc1 = jnp.where(i1 == e, oe, acc1)
        acc2 = jnp.where(i2 == e, oe, acc2)
    out_ref[...] = jnp.stack([acc1, acc2], axis=1)


def kernel(x, gate_w, gate_b, expert_w, expert_b):
    B, S, EMB = x.shape
    NE, _, HID = expert_w.shape
    T = B * S
    BT = 512
    x2d = x.reshape(T, EMB)
    gw = jnp.pad(gate_w, ((0, 0), (0, 128 - NE)))
    gb = jnp.pad(gate_b, (0, 128 - NE)).reshape(1, 128)
    ew16 = expert_w.astype(jnp.bfloat16)

    topw, out = pl.pallas_call(
        functools.partial(_fused_body, NE),
        grid=(T // BT,),
        in_specs=[
            pl.BlockSpec((BT, EMB), lambda t: (t, 0)),
            pl.BlockSpec((EMB, 128), lambda t: (0, 0)),
            pl.BlockSpec((1, 128), lambda t: (0, 0)),
            pl.BlockSpec((NE, EMB, HID), lambda t: (0, 0, 0)),
            pl.BlockSpec((NE, HID), lambda t: (0, 0)),
        ],
        out_specs=[
            pl.BlockSpec((BT, 2), lambda t: (t, 0)),
            pl.BlockSpec((BT, 2, HID), lambda t: (t, 0, 0)),
        ],
        out_shape=[
            jax.ShapeDtypeStruct((T, 2), jnp.float32),
            jax.ShapeDtypeStruct((T, 2, HID), jnp.float32),
        ],
    )(x2d, gw, gb, ew16, expert_b)

    return topw.reshape(B, S, 2), out.reshape(B, S, 2, HID)
